# baseline (speedup 1.0000x reference)
import functools

import jax
import jax.numpy as jnp
from jax import lax
from jax.experimental import pallas as pl
from jax.experimental.pallas import tpu as pltpu
from jax.experimental.pallas import tpu_sc as plsc

NC = 2
NS = 16
NW = NC * NS
LW = 16

CH = 80
ACH = 40
RB = 5
def _npad(n):
    return ((n + 2047) // 2048) * 2048


def _mesh():
    return plsc.VectorSubcoreMesh(core_axis_name="c", subcore_axis_name="s")


def _degrees_sc(src, dst, n):
    nch = src.shape[1]
    npad = _npad(n)
    npt = npad // NS

    def body(src_hbm, dst_hbm, out_hbm, sidx, didx, ones_s, ones_d, zbuf, acc,
             ss0, ss1, sd0, sd1):
        c = lax.axis_index("c")
        s = lax.axis_index("s")
        wid = s * NC + c

        pltpu.sync_copy(src_hbm.at[wid], sidx)
        pltpu.sync_copy(dst_hbm.at[wid], didx)

        lane = lax.iota(jnp.int32, 16)
        one = jnp.ones((LW,), jnp.float32)
        zero = jnp.zeros((LW,), jnp.float32)

        def fill(i, carry):
            ones_s[i] = jnp.where(lane < 8, one, zero)
            ones_d[i] = jnp.where(lane < 8, zero, one)
            zbuf[i] = zero
            return carry

        lax.fori_loop(0, CH, fill, 0)

        r0 = s * npt

        def zrow(i, carry):
            pltpu.sync_copy(zbuf, acc.at[pl.ds(r0 + i * CH, CH)])
            return carry

        lax.fori_loop(0, npt // CH, zrow, 0)
        plsc.subcore_barrier()

        ssem = (ss0, ss1)
        dsem = (sd0, sd1)

        def fire(gi, b):
            pltpu.async_copy(ones_s, acc.at[sidx.at[gi]], ssem[b], add=True)
            pltpu.async_copy(ones_d, acc.at[didx.at[gi]], dsem[b], add=True)

        def wait(gi, b):
            pltpu.make_async_copy(ones_s, acc.at[sidx.at[gi]], ssem[b]).wait()
            pltpu.make_async_copy(ones_d, acc.at[didx.at[gi]], dsem[b]).wait()

        fire(0, 0)
        fire(1, 1)

        lp = (nch - 2) // 2

        def pair(gg, carry):
            for b in (0, 1):
                gi = 2 * gg + b
                wait(gi, b)
                fire(gi + 2, b)
            return carry

        lax.fori_loop(0, lp, pair, 0)
        for gi in range(2 * lp, nch):
            b = gi % 2
            wait(gi, b)
            if gi + 2 < nch:
                fire(gi + 2, b)
        plsc.subcore_barrier()

        pltpu.sync_copy(acc.at[pl.ds(r0, npt)], out_hbm.at[c, pl.ds(r0, npt)])

    f = pl.kernel(
        body,
        out_type=jax.ShapeDtypeStruct((NC, npad, LW), jnp.float32),
        mesh=_mesh(),
        compiler_params=pltpu.CompilerParams(use_tc_tiling_on_sc=False),
        scratch_types=[
            pltpu.VMEM((nch, CH), jnp.int32),
            pltpu.VMEM((nch, CH), jnp.int32),
            pltpu.VMEM((CH, LW), jnp.float32),
            pltpu.VMEM((CH, LW), jnp.float32),
            pltpu.VMEM((CH, LW), jnp.float32),
            pltpu.VMEM_SHARED((npad, LW), jnp.float32),
            pltpu.SemaphoreType.DMA,
            pltpu.SemaphoreType.DMA,
            pltpu.SemaphoreType.DMA,
            pltpu.SemaphoreType.DMA,
        ],
    )
    return f(src, dst)


def _aggregate_sc(t, src, dst, n, d, tc_tiling):
    e = src.shape[0]
    ept = e // NW
    nch = ept // ACH
    npad = _npad(n)
    npt = npad // NS
    assert nch % RB == 0 and npt % ACH == 0

    def body(t_hbm, src_hbm, dst_hbm, out_hbm, sidx, didx, rows, acc,
             gsems, dsems, ssems):
        c = lax.axis_index("c")
        s = lax.axis_index("s")
        wid = s * NC + c
        e0 = wid * ept

        pltpu.sync_copy(src_hbm.at[pl.ds(e0, ept)], sidx)

        def zfill(i, carry):
            for j in range(d // LW):
                rows[0][i, pl.ds(j * LW, LW)] = jnp.zeros((LW,), jnp.float32)
            return carry

        lax.fori_loop(0, ACH, zfill, 0)

        r0 = s * npt

        def zrow(i, carry):
            pltpu.sync_copy(rows[0], acc.at[pl.ds(r0 + i * ACH, ACH)])
            return carry

        lax.fori_loop(0, npt // ACH, zrow, 0)
        plsc.subcore_barrier()

        def fire_g(gi, b):
            pltpu.async_copy(t_hbm.at[sidx.at[pl.ds(gi * ACH, ACH)]], rows[b],
                             gsems[b])
            pltpu.async_copy(dst_hbm.at[pl.ds(e0 + gi * ACH, ACH)], didx[b],
                             dsems[b])

        def wait_g(gi, b):
            pltpu.make_async_copy(t_hbm.at[sidx.at[pl.ds(gi * ACH, ACH)]],
                                  rows[b], gsems[b]).wait()
            pltpu.make_async_copy(dst_hbm.at[pl.ds(e0 + gi * ACH, ACH)],
                                  didx[b], dsems[b]).wait()

        def fire_s(gi, b):
            pltpu.async_copy(rows[b], acc.at[didx[b]], ssems[b], add=True)

        def wait_s(gi, b):
            pltpu.make_async_copy(rows[b], acc.at[didx[b]], ssems[b]).wait()

        for b in range(RB):
            fire_g(b, b)

        def grp(gg, carry):
            g = RB * gg
            for b in range(RB):
                wait_g(g + b, b)
                fire_s(g + b, b)
            for b in range(RB):
                wait_s(g + b, b)
                fire_g(g + RB + b, b)
            return carry

        lax.fori_loop(0, nch // RB - 1, grp, 0)
        ge = nch - RB
        for b in range(RB):
            wait_g(ge + b, b)
            fire_s(ge + b, b)
        for b in range(RB):
            wait_s(ge + b, b)
        plsc.subcore_barrier()

        pltpu.sync_copy(acc.at[pl.ds(r0, npt)], out_hbm.at[c, pl.ds(r0, npt)])

    def wrapped(t_hbm, src_hbm, dst_hbm, out_hbm, sidx, *rest):
        didx = rest[:RB]
        rows = rest[RB:2 * RB]
        acc = rest[2 * RB]
        gsems = rest[2 * RB + 1:3 * RB + 1]
        dsems = rest[3 * RB + 1:4 * RB + 1]
        ssems = rest[4 * RB + 1:]
        return body(t_hbm, src_hbm, dst_hbm, out_hbm, sidx, didx, rows, acc,
                    gsems, dsems, ssems)

    f = pl.kernel(
        wrapped,
        out_type=jax.ShapeDtypeStruct((NC, npad, d), jnp.float32),
        mesh=_mesh(),
        compiler_params=pltpu.CompilerParams(use_tc_tiling_on_sc=tc_tiling),
        scratch_types=[pltpu.VMEM((ept,), jnp.int32)]
          + [pltpu.VMEM((ACH,), jnp.int32) for _ in range(RB)]
          + [pltpu.VMEM((ACH, d), jnp.float32) for _ in range(RB)]
          + [pltpu.VMEM_SHARED((npad, d), jnp.float32)]
          + [pltpu.SemaphoreType.DMA for _ in range(3 * RB)],
    )
    return f(t, src, dst)


def _k1_body(x_ref, w1_ref, degp_ref, t1_ref, ns_ref, nd_ref):
    dp = degp_ref[...]
    deg_out = dp[0, :, 0] + dp[1, :, 0]
    deg_in = dp[0, :, 8] + dp[1, :, 8]
    ns = jnp.where(deg_out > 0, lax.rsqrt(jnp.maximum(deg_out, 1.0)), 0.0)
    nd = jnp.where(deg_in > 0, lax.rsqrt(jnp.maximum(deg_in, 1.0)), 0.0)
    t1 = jnp.dot(x_ref[...], w1_ref[...], preferred_element_type=jnp.float32)
    t1_ref[...] = t1 * ns[:, None]
    ns_ref[...] = ns[:, None]
    nd_ref[...] = nd[:, None]


def _k3_body(ap_ref, nd_ref, b1_ref, w2_ref, ns_ref, t2_ref):
    a = ap_ref[0] + ap_ref[1]
    h = jnp.maximum(a * nd_ref[...] + b1_ref[...], 0.0)
    t2 = jnp.dot(h, w2_ref[...], preferred_element_type=jnp.float32)
    t2_ref[...] = t2 * ns_ref[...]


def _k5_body(ap_ref, nd_ref, b2_ref, o_ref):
    a = ap_ref[0] + ap_ref[1]
    o_ref[...] = a * nd_ref[...] + b2_ref[...]


def kernel(x, edge_index, W1, b1, W2, b2):
    n, d_in = x.shape
    d_h = W1.shape[1]
    d_out = W2.shape[1]
    e = edge_index.shape[1]
    erd = edge_index.reshape(2, NW, e // NW // CH, CH)

    degp = _degrees_sc(erd[0], erd[1], n)

    R = 1000
    grid = (n // R,)

    t1, nsrc, ndst = pl.pallas_call(
        _k1_body,
        grid=grid,
        in_specs=[
            pl.BlockSpec((R, d_in), lambda i: (i, 0)),
            pl.BlockSpec((d_in, d_h), lambda i: (0, 0)),
            pl.BlockSpec((NC, R, LW), lambda i: (0, i, 0)),
        ],
        out_specs=[
            pl.BlockSpec((R, d_h), lambda i: (i, 0)),
            pl.BlockSpec((R, 1), lambda i: (i, 0)),
            pl.BlockSpec((R, 1), lambda i: (i, 0)),
        ],
        out_shape=[
            jax.ShapeDtypeStruct((n, d_h), jnp.float32),
            jax.ShapeDtypeStruct((n, 1), jnp.float32),
            jax.ShapeDtypeStruct((n, 1), jnp.float32),
        ],
    )(x, W1, degp)

    agg1 = _aggregate_sc(t1, edge_index[0], edge_index[1], n, d_h, tc_tiling=True)

    t2 = pl.pallas_call(
        _k3_body,
        grid=grid,
        in_specs=[
            pl.BlockSpec((NC, R, d_h), lambda i: (0, i, 0)),
            pl.BlockSpec((R, 1), lambda i: (i, 0)),
            pl.BlockSpec((1, d_h), lambda i: (0, 0)),
            pl.BlockSpec((d_h, d_out), lambda i: (0, 0)),
            pl.BlockSpec((R, 1), lambda i: (i, 0)),
        ],
        out_specs=pl.BlockSpec((R, d_out), lambda i: (i, 0)),
        out_shape=jax.ShapeDtypeStruct((n, d_out), jnp.float32),
    )(agg1, ndst, b1[None, :], W2, nsrc)

    agg2 = _aggregate_sc(t2, edge_index[0], edge_index[1], n, d_out, tc_tiling=False)

    out = pl.pallas_call(
        _k5_body,
        grid=grid,
        in_specs=[
            pl.BlockSpec((NC, R, d_out), lambda i: (0, i, 0)),
            pl.BlockSpec((R, 1), lambda i: (i, 0)),
            pl.BlockSpec((1, d_out), lambda i: (0, 0)),
        ],
        out_specs=pl.BlockSpec((R, d_out), lambda i: (i, 0)),
        out_shape=jax.ShapeDtypeStruct((n, d_out), jnp.float32),
    )(agg2, ndst, b2[None, :])

    return out


# speedup vs baseline: 1.0169x; 1.0169x over previous
import functools

import jax
import jax.numpy as jnp
from jax import lax
from jax.experimental import pallas as pl
from jax.experimental.pallas import tpu as pltpu
from jax.experimental.pallas import tpu_sc as plsc

NC = 2
NS = 16
NW = NC * NS
LW = 16

CH = 80
ACH = 40
RB = 5
def _npad(n):
    return ((n + 2047) // 2048) * 2048


def _mesh():
    return plsc.VectorSubcoreMesh(core_axis_name="c", subcore_axis_name="s")


def _degrees_sc(edge_index, n):
    e = edge_index.shape[0] // 2
    ept = e // NW
    nch = ept // ACH
    npad = _npad(n)
    npt = npad // NS
    assert nch % RB == 0 and npt % ACH == 0

    def body(ei_hbm, out_hbm, ones_s, ones_d, zbuf, sbufs, dbufs, acc,
             isems, ssems, dsems):
        c = lax.axis_index("c")
        s = lax.axis_index("s")
        wid = s * NC + c
        e0 = wid * ept

        lane = lax.iota(jnp.int32, 16)
        one = jnp.ones((LW,), jnp.float32)
        zero = jnp.zeros((LW,), jnp.float32)

        def fill(i, carry):
            ones_s[i] = jnp.where(lane < 8, one, zero)
            ones_d[i] = jnp.where(lane < 8, zero, one)
            zbuf[i] = zero
            return carry

        lax.fori_loop(0, ACH, fill, 0)

        r0 = s * npt

        def zrow(i, carry):
            pltpu.sync_copy(zbuf, acc.at[pl.ds(r0 + i * ACH, ACH)])
            return carry

        lax.fori_loop(0, npt // ACH, zrow, 0)
        plsc.subcore_barrier()

        def fire_i(gi, b):
            pltpu.async_copy(ei_hbm.at[pl.ds(e0 + gi * ACH, ACH)], sbufs[b],
                             isems[b])
            pltpu.async_copy(ei_hbm.at[pl.ds(e + e0 + gi * ACH, ACH)], dbufs[b],
                             isems[b])

        def wait_i(gi, b):
            pltpu.make_async_copy(ei_hbm.at[pl.ds(e0 + gi * ACH, ACH)],
                                  sbufs[b], isems[b]).wait()
            pltpu.make_async_copy(ei_hbm.at[pl.ds(e + e0 + gi * ACH, ACH)],
                                  dbufs[b], isems[b]).wait()

        def fire_s(gi, b):
            pltpu.async_copy(ones_s, acc.at[sbufs[b]], ssems[b], add=True)
            pltpu.async_copy(ones_d, acc.at[dbufs[b]], dsems[b], add=True)

        def wait_s(gi, b):
            pltpu.make_async_copy(ones_s, acc.at[sbufs[b]], ssems[b]).wait()
            pltpu.make_async_copy(ones_d, acc.at[dbufs[b]], dsems[b]).wait()

        for b in range(RB):
            fire_i(b, b)

        def grp(gg, carry):
            g = RB * gg
            for b in range(RB):
                wait_i(g + b, b)
                fire_s(g + b, b)
            for b in range(RB):
                wait_s(g + b, b)
                fire_i(g + RB + b, b)
            return carry

        lax.fori_loop(0, nch // RB - 1, grp, 0)
        ge = nch - RB
        for b in range(RB):
            wait_i(ge + b, b)
            fire_s(ge + b, b)
        for b in range(RB):
            wait_s(ge + b, b)
        plsc.subcore_barrier()

        pltpu.sync_copy(acc.at[pl.ds(r0, npt)], out_hbm.at[c, pl.ds(r0, npt)])

    def wrapped(ei_hbm, out_hbm, ones_s, ones_d, zbuf, *rest):
        sbufs = rest[:RB]
        dbufs = rest[RB:2 * RB]
        acc = rest[2 * RB]
        isems = rest[2 * RB + 1:3 * RB + 1]
        ssems = rest[3 * RB + 1:4 * RB + 1]
        dsems = rest[4 * RB + 1:]
        return body(ei_hbm, out_hbm, ones_s, ones_d, zbuf, sbufs, dbufs, acc,
                    isems, ssems, dsems)

    f = pl.kernel(
        wrapped,
        out_type=jax.ShapeDtypeStruct((NC, npad, LW), jnp.float32),
        mesh=_mesh(),
        compiler_params=pltpu.CompilerParams(use_tc_tiling_on_sc=False),
        scratch_types=[
            pltpu.VMEM((ACH, LW), jnp.float32),
            pltpu.VMEM((ACH, LW), jnp.float32),
            pltpu.VMEM((ACH, LW), jnp.float32),
        ] + [pltpu.VMEM((ACH,), jnp.int32) for _ in range(2 * RB)]
          + [pltpu.VMEM_SHARED((npad, LW), jnp.float32)]
          + [pltpu.SemaphoreType.DMA for _ in range(3 * RB)],
    )
    return f(edge_index)


def _aggregate_sc(t, edge_index, n, d, tc_tiling):
    e = edge_index.shape[0] // 2
    ept = e // NW
    nch = ept // ACH
    npad = _npad(n)
    npt = npad // NS
    assert nch % RB == 0 and npt % ACH == 0

    def body(t_hbm, ei_hbm, out_hbm, sidx, didx, rows, acc,
             gsems, dsems, ssems):
        c = lax.axis_index("c")
        s = lax.axis_index("s")
        wid = s * NC + c
        e0 = wid * ept

        pltpu.sync_copy(ei_hbm.at[pl.ds(e0, ept)], sidx)

        def zfill(i, carry):
            for j in range(d // LW):
                rows[0][i, pl.ds(j * LW, LW)] = jnp.zeros((LW,), jnp.float32)
            return carry

        lax.fori_loop(0, ACH, zfill, 0)

        r0 = s * npt

        def zrow(i, carry):
            pltpu.sync_copy(rows[0], acc.at[pl.ds(r0 + i * ACH, ACH)])
            return carry

        lax.fori_loop(0, npt // ACH, zrow, 0)
        plsc.subcore_barrier()

        def fire_g(gi, b):
            pltpu.async_copy(t_hbm.at[sidx.at[pl.ds(gi * ACH, ACH)]], rows[b],
                             gsems[b])
            pltpu.async_copy(ei_hbm.at[pl.ds(e + e0 + gi * ACH, ACH)], didx[b],
                             dsems[b])

        def wait_g(gi, b):
            pltpu.make_async_copy(t_hbm.at[sidx.at[pl.ds(gi * ACH, ACH)]],
                                  rows[b], gsems[b]).wait()
            pltpu.make_async_copy(ei_hbm.at[pl.ds(e + e0 + gi * ACH, ACH)],
                                  didx[b], dsems[b]).wait()

        def fire_s(gi, b):
            pltpu.async_copy(rows[b], acc.at[didx[b]], ssems[b], add=True)

        def wait_s(gi, b):
            pltpu.make_async_copy(rows[b], acc.at[didx[b]], ssems[b]).wait()

        for b in range(RB):
            fire_g(b, b)

        def grp(gg, carry):
            g = RB * gg
            for b in range(RB):
                wait_g(g + b, b)
                fire_s(g + b, b)
            for b in range(RB):
                wait_s(g + b, b)
                fire_g(g + RB + b, b)
            return carry

        lax.fori_loop(0, nch // RB - 1, grp, 0)
        ge = nch - RB
        for b in range(RB):
            wait_g(ge + b, b)
            fire_s(ge + b, b)
        for b in range(RB):
            wait_s(ge + b, b)
        plsc.subcore_barrier()

        pltpu.sync_copy(acc.at[pl.ds(r0, npt)], out_hbm.at[c, pl.ds(r0, npt)])

    def wrapped(t_hbm, ei_hbm, out_hbm, sidx, *rest):
        didx = rest[:RB]
        rows = rest[RB:2 * RB]
        acc = rest[2 * RB]
        gsems = rest[2 * RB + 1:3 * RB + 1]
        dsems = rest[3 * RB + 1:4 * RB + 1]
        ssems = rest[4 * RB + 1:]
        return body(t_hbm, ei_hbm, out_hbm, sidx, didx, rows, acc,
                    gsems, dsems, ssems)

    f = pl.kernel(
        wrapped,
        out_type=jax.ShapeDtypeStruct((NC, npad, d), jnp.float32),
        mesh=_mesh(),
        compiler_params=pltpu.CompilerParams(use_tc_tiling_on_sc=tc_tiling),
        scratch_types=[pltpu.VMEM((ept,), jnp.int32)]
          + [pltpu.VMEM((ACH,), jnp.int32) for _ in range(RB)]
          + [pltpu.VMEM((ACH, d), jnp.float32) for _ in range(RB)]
          + [pltpu.VMEM_SHARED((npad, d), jnp.float32)]
          + [pltpu.SemaphoreType.DMA for _ in range(3 * RB)],
    )
    return f(t, edge_index)


def _k1_body(x_ref, w1_ref, degp_ref, t1_ref, ns_ref, nd_ref):
    dp = degp_ref[...]
    deg_out = dp[0, :, 0] + dp[1, :, 0]
    deg_in = dp[0, :, 8] + dp[1, :, 8]
    ns = jnp.where(deg_out > 0, lax.rsqrt(jnp.maximum(deg_out, 1.0)), 0.0)
    nd = jnp.where(deg_in > 0, lax.rsqrt(jnp.maximum(deg_in, 1.0)), 0.0)
    t1 = jnp.dot(x_ref[...], w1_ref[...], preferred_element_type=jnp.float32)
    t1_ref[...] = t1 * ns[:, None]
    ns_ref[...] = ns[:, None]
    nd_ref[...] = nd[:, None]


def _k3_body(ap_ref, nd_ref, b1_ref, w2_ref, ns_ref, t2_ref):
    a = ap_ref[0] + ap_ref[1]
    h = jnp.maximum(a * nd_ref[...] + b1_ref[...], 0.0)
    t2 = jnp.dot(h, w2_ref[...], preferred_element_type=jnp.float32)
    t2_ref[...] = t2 * ns_ref[...]


def _k5_body(ap_ref, nd_ref, b2_ref, o_ref):
    a = ap_ref[0] + ap_ref[1]
    o_ref[...] = a * nd_ref[...] + b2_ref[...]


def kernel(x, edge_index, W1, b1, W2, b2):
    n, d_in = x.shape
    d_h = W1.shape[1]
    d_out = W2.shape[1]
    ei_flat = edge_index.reshape(-1)

    degp = _degrees_sc(ei_flat, n)

    R = 1000
    grid = (n // R,)

    t1, nsrc, ndst = pl.pallas_call(
        _k1_body,
        grid=grid,
        in_specs=[
            pl.BlockSpec((R, d_in), lambda i: (i, 0)),
            pl.BlockSpec((d_in, d_h), lambda i: (0, 0)),
            pl.BlockSpec((NC, R, LW), lambda i: (0, i, 0)),
        ],
        out_specs=[
            pl.BlockSpec((R, d_h), lambda i: (i, 0)),
            pl.BlockSpec((R, 1), lambda i: (i, 0)),
            pl.BlockSpec((R, 1), lambda i: (i, 0)),
        ],
        out_shape=[
            jax.ShapeDtypeStruct((n, d_h), jnp.float32),
            jax.ShapeDtypeStruct((n, 1), jnp.float32),
            jax.ShapeDtypeStruct((n, 1), jnp.float32),
        ],
    )(x, W1, degp)

    agg1 = _aggregate_sc(t1, ei_flat, n, d_h, tc_tiling=True)

    t2 = pl.pallas_call(
        _k3_body,
        grid=grid,
        in_specs=[
            pl.BlockSpec((NC, R, d_h), lambda i: (0, i, 0)),
            pl.BlockSpec((R, 1), lambda i: (i, 0)),
            pl.BlockSpec((1, d_h), lambda i: (0, 0)),
            pl.BlockSpec((d_h, d_out), lambda i: (0, 0)),
            pl.BlockSpec((R, 1), lambda i: (i, 0)),
        ],
        out_specs=pl.BlockSpec((R, d_out), lambda i: (i, 0)),
        out_shape=jax.ShapeDtypeStruct((n, d_out), jnp.float32),
    )(agg1, ndst, b1[None, :], W2, nsrc)

    agg2 = _aggregate_sc(t2, ei_flat, n, d_out, tc_tiling=False)

    out = pl.pallas_call(
        _k5_body,
        grid=grid,
        in_specs=[
            pl.BlockSpec((NC, R, d_out), lambda i: (0, i, 0)),
            pl.BlockSpec((R, 1), lambda i: (i, 0)),
            pl.BlockSpec((1, d_out), lambda i: (0, 0)),
        ],
        out_specs=pl.BlockSpec((R, d_out), lambda i: (i, 0)),
        out_shape=jax.ShapeDtypeStruct((n, d_out), jnp.float32),
    )(agg2, ndst, b2[None, :])

    return out


# speedup vs baseline: 1.0615x; 1.0439x over previous
import functools

import jax
import jax.numpy as jnp
from jax import lax
from jax.experimental import pallas as pl
from jax.experimental.pallas import tpu as pltpu
from jax.experimental.pallas import tpu_sc as plsc

NC = 2
NS = 16
NW = NC * NS
LW = 16

CH = 80
ACH = 40
RB = 5
def _npad(n):
    return ((n + 2047) // 2048) * 2048


def _mesh():
    return plsc.VectorSubcoreMesh(core_axis_name="c", subcore_axis_name="s")


def _degrees_sc(edge_index, n):
    e = edge_index.shape[0] // 2
    ept = e // NW
    nch = ept // CH
    npad = _npad(n)
    npt = npad // NS
    assert nch % RB == 0 and npt % CH == 0

    def body(ei_hbm, out_hbm, ones_s, ones_d, zbuf, sbufs, dbufs, acc,
             isems, ssems, dsems):
        c = lax.axis_index("c")
        s = lax.axis_index("s")
        wid = s * NC + c
        e0 = wid * ept

        lane = lax.iota(jnp.int32, 16)
        one = jnp.ones((LW,), jnp.float32)
        zero = jnp.zeros((LW,), jnp.float32)

        def fill(i, carry):
            ones_s[i] = jnp.where(lane < 8, one, zero)
            ones_d[i] = jnp.where(lane < 8, zero, one)
            zbuf[i] = zero
            return carry

        lax.fori_loop(0, CH, fill, 0)

        r0 = s * npt

        def zrow(i, carry):
            pltpu.sync_copy(zbuf, acc.at[pl.ds(r0 + i * CH, CH)])
            return carry

        lax.fori_loop(0, npt // CH, zrow, 0)
        plsc.subcore_barrier()

        def fire_i(gi, b):
            pltpu.async_copy(ei_hbm.at[pl.ds(e0 + gi * CH, CH)], sbufs[b],
                             isems[b])
            pltpu.async_copy(ei_hbm.at[pl.ds(e + e0 + gi * CH, CH)], dbufs[b],
                             isems[b])

        def wait_i(gi, b):
            pltpu.make_async_copy(ei_hbm.at[pl.ds(e0 + gi * CH, CH)],
                                  sbufs[b], isems[b]).wait()
            pltpu.make_async_copy(ei_hbm.at[pl.ds(e + e0 + gi * CH, CH)],
                                  dbufs[b], isems[b]).wait()

        def fire_s(gi, b):
            pltpu.async_copy(ones_s, acc.at[sbufs[b]], ssems[b], add=True)
            pltpu.async_copy(ones_d, acc.at[dbufs[b]], dsems[b], add=True)

        def wait_s(gi, b):
            pltpu.make_async_copy(ones_s, acc.at[sbufs[b]], ssems[b]).wait()
            pltpu.make_async_copy(ones_d, acc.at[dbufs[b]], dsems[b]).wait()

        for b in range(RB):
            fire_i(b, b)

        def grp(gg, carry):
            g = RB * gg
            for b in range(RB):
                wait_i(g + b, b)
                fire_s(g + b, b)
            for b in range(RB):
                wait_s(g + b, b)
                fire_i(g + RB + b, b)
            return carry

        lax.fori_loop(0, nch // RB - 1, grp, 0)
        ge = nch - RB
        for b in range(RB):
            wait_i(ge + b, b)
            fire_s(ge + b, b)
        for b in range(RB):
            wait_s(ge + b, b)
        plsc.subcore_barrier()

        pltpu.sync_copy(acc.at[pl.ds(r0, npt)], out_hbm.at[c, pl.ds(r0, npt)])

    def wrapped(ei_hbm, out_hbm, ones_s, ones_d, zbuf, *rest):
        sbufs = rest[:RB]
        dbufs = rest[RB:2 * RB]
        acc = rest[2 * RB]
        isems = rest[2 * RB + 1:3 * RB + 1]
        ssems = rest[3 * RB + 1:4 * RB + 1]
        dsems = rest[4 * RB + 1:]
        return body(ei_hbm, out_hbm, ones_s, ones_d, zbuf, sbufs, dbufs, acc,
                    isems, ssems, dsems)

    f = pl.kernel(
        wrapped,
        out_type=jax.ShapeDtypeStruct((NC, npad, LW), jnp.float32),
        mesh=_mesh(),
        compiler_params=pltpu.CompilerParams(use_tc_tiling_on_sc=False),
        scratch_types=[
            pltpu.VMEM((CH, LW), jnp.float32),
            pltpu.VMEM((CH, LW), jnp.float32),
            pltpu.VMEM((CH, LW), jnp.float32),
        ] + [pltpu.VMEM((CH,), jnp.int32) for _ in range(2 * RB)]
          + [pltpu.VMEM_SHARED((npad, LW), jnp.float32)]
          + [pltpu.SemaphoreType.DMA for _ in range(3 * RB)],
    )
    return f(edge_index)


def _aggregate_sc(t, edge_index, n, d, tc_tiling):
    e = edge_index.shape[0] // 2
    ept = e // NW
    nch = ept // ACH
    npad = _npad(n)
    npt = npad // NS
    assert nch % RB == 0 and npt % ACH == 0

    def body(t_hbm, ei_hbm, out_hbm, sidx, didx, rows, acc,
             gsems, dsems, ssems):
        c = lax.axis_index("c")
        s = lax.axis_index("s")
        wid = s * NC + c
        e0 = wid * ept

        pltpu.sync_copy(ei_hbm.at[pl.ds(e0, ept)], sidx)

        def zfill(i, carry):
            for j in range(d // LW):
                rows[0][i, pl.ds(j * LW, LW)] = jnp.zeros((LW,), jnp.float32)
            return carry

        lax.fori_loop(0, ACH, zfill, 0)

        r0 = s * npt

        def zrow(i, carry):
            pltpu.sync_copy(rows[0], acc.at[pl.ds(r0 + i * ACH, ACH)])
            return carry

        lax.fori_loop(0, npt // ACH, zrow, 0)
        plsc.subcore_barrier()

        def fire_g(gi, b):
            pltpu.async_copy(t_hbm.at[sidx.at[pl.ds(gi * ACH, ACH)]], rows[b],
                             gsems[b])
            pltpu.async_copy(ei_hbm.at[pl.ds(e + e0 + gi * ACH, ACH)], didx[b],
                             dsems[b])

        def wait_g(gi, b):
            pltpu.make_async_copy(t_hbm.at[sidx.at[pl.ds(gi * ACH, ACH)]],
                                  rows[b], gsems[b]).wait()
            pltpu.make_async_copy(ei_hbm.at[pl.ds(e + e0 + gi * ACH, ACH)],
                                  didx[b], dsems[b]).wait()

        def fire_s(gi, b):
            pltpu.async_copy(rows[b], acc.at[didx[b]], ssems[b], add=True)

        def wait_s(gi, b):
            pltpu.make_async_copy(rows[b], acc.at[didx[b]], ssems[b]).wait()

        for b in range(RB):
            fire_g(b, b)

        def grp(gg, carry):
            g = RB * gg
            for b in range(RB):
                wait_g(g + b, b)
                fire_s(g + b, b)
            for b in range(RB):
                wait_s(g + b, b)
                fire_g(g + RB + b, b)
            return carry

        lax.fori_loop(0, nch // RB - 1, grp, 0)
        ge = nch - RB
        for b in range(RB):
            wait_g(ge + b, b)
            fire_s(ge + b, b)
        for b in range(RB):
            wait_s(ge + b, b)
        plsc.subcore_barrier()

        pltpu.sync_copy(acc.at[pl.ds(r0, npt)], out_hbm.at[c, pl.ds(r0, npt)])

    def wrapped(t_hbm, ei_hbm, out_hbm, sidx, *rest):
        didx = rest[:RB]
        rows = rest[RB:2 * RB]
        acc = rest[2 * RB]
        gsems = rest[2 * RB + 1:3 * RB + 1]
        dsems = rest[3 * RB + 1:4 * RB + 1]
        ssems = rest[4 * RB + 1:]
        return body(t_hbm, ei_hbm, out_hbm, sidx, didx, rows, acc,
                    gsems, dsems, ssems)

    f = pl.kernel(
        wrapped,
        out_type=jax.ShapeDtypeStruct((NC, npad, d), jnp.float32),
        mesh=_mesh(),
        compiler_params=pltpu.CompilerParams(use_tc_tiling_on_sc=tc_tiling),
        scratch_types=[pltpu.VMEM((ept,), jnp.int32)]
          + [pltpu.VMEM((ACH,), jnp.int32) for _ in range(RB)]
          + [pltpu.VMEM((ACH, d), jnp.float32) for _ in range(RB)]
          + [pltpu.VMEM_SHARED((npad, d), jnp.float32)]
          + [pltpu.SemaphoreType.DMA for _ in range(3 * RB)],
    )
    return f(t, edge_index)


def _k1_body(x_ref, w1_ref, degp_ref, t1_ref, ns_ref, nd_ref):
    dp = degp_ref[...]
    deg_out = dp[0, :, 0] + dp[1, :, 0]
    deg_in = dp[0, :, 8] + dp[1, :, 8]
    ns = jnp.where(deg_out > 0, lax.rsqrt(jnp.maximum(deg_out, 1.0)), 0.0)
    nd = jnp.where(deg_in > 0, lax.rsqrt(jnp.maximum(deg_in, 1.0)), 0.0)
    t1 = jnp.dot(x_ref[...], w1_ref[...], preferred_element_type=jnp.float32)
    t1_ref[...] = t1 * ns[:, None]
    ns_ref[...] = ns[:, None]
    nd_ref[...] = nd[:, None]


def _k3_body(ap_ref, nd_ref, b1_ref, w2_ref, ns_ref, t2_ref):
    a = ap_ref[0] + ap_ref[1]
    h = jnp.maximum(a * nd_ref[...] + b1_ref[...], 0.0)
    t2 = jnp.dot(h, w2_ref[...], preferred_element_type=jnp.float32)
    t2_ref[...] = t2 * ns_ref[...]


def _k5_body(ap_ref, nd_ref, b2_ref, o_ref):
    a = ap_ref[0] + ap_ref[1]
    o_ref[...] = a * nd_ref[...] + b2_ref[...]


def kernel(x, edge_index, W1, b1, W2, b2):
    n, d_in = x.shape
    d_h = W1.shape[1]
    d_out = W2.shape[1]
    ei_flat = edge_index.reshape(-1)

    degp = _degrees_sc(ei_flat, n)

    R = 1000
    grid = (n // R,)

    t1, nsrc, ndst = pl.pallas_call(
        _k1_body,
        grid=grid,
        in_specs=[
            pl.BlockSpec((R, d_in), lambda i: (i, 0)),
            pl.BlockSpec((d_in, d_h), lambda i: (0, 0)),
            pl.BlockSpec((NC, R, LW), lambda i: (0, i, 0)),
        ],
        out_specs=[
            pl.BlockSpec((R, d_h), lambda i: (i, 0)),
            pl.BlockSpec((R, 1), lambda i: (i, 0)),
            pl.BlockSpec((R, 1), lambda i: (i, 0)),
        ],
        out_shape=[
            jax.ShapeDtypeStruct((n, d_h), jnp.float32),
            jax.ShapeDtypeStruct((n, 1), jnp.float32),
            jax.ShapeDtypeStruct((n, 1), jnp.float32),
        ],
    )(x, W1, degp)

    agg1 = _aggregate_sc(t1, ei_flat, n, d_h, tc_tiling=True)

    t2 = pl.pallas_call(
        _k3_body,
        grid=grid,
        in_specs=[
            pl.BlockSpec((NC, R, d_h), lambda i: (0, i, 0)),
            pl.BlockSpec((R, 1), lambda i: (i, 0)),
            pl.BlockSpec((1, d_h), lambda i: (0, 0)),
            pl.BlockSpec((d_h, d_out), lambda i: (0, 0)),
            pl.BlockSpec((R, 1), lambda i: (i, 0)),
        ],
        out_specs=pl.BlockSpec((R, d_out), lambda i: (i, 0)),
        out_shape=jax.ShapeDtypeStruct((n, d_out), jnp.float32),
    )(agg1, ndst, b1[None, :], W2, nsrc)

    agg2 = _aggregate_sc(t2, ei_flat, n, d_out, tc_tiling=False)

    out = pl.pallas_call(
        _k5_body,
        grid=grid,
        in_specs=[
            pl.BlockSpec((NC, R, d_out), lambda i: (0, i, 0)),
            pl.BlockSpec((R, 1), lambda i: (i, 0)),
            pl.BlockSpec((1, d_out), lambda i: (0, 0)),
        ],
        out_specs=pl.BlockSpec((R, d_out), lambda i: (i, 0)),
        out_shape=jax.ShapeDtypeStruct((n, d_out), jnp.float32),
    )(agg2, ndst, b2[None, :])

    return out


# speedup vs baseline: 1.1268x; 1.0616x over previous
import functools

import jax
import jax.numpy as jnp
from jax import lax
from jax.experimental import pallas as pl
from jax.experimental.pallas import tpu as pltpu
from jax.experimental.pallas import tpu_sc as plsc

NC = 2
NS = 16
NW = NC * NS
LW = 16

CH = 80
ACH = 40
RB = 5
def _npad(n):
    return ((n + 2047) // 2048) * 2048


def _mesh():
    return plsc.VectorSubcoreMesh(core_axis_name="c", subcore_axis_name="s")


def _degrees_sc(edge_index, n):
    e = edge_index.shape[0] // 2
    ept = e // NW
    nch = ept // CH
    npad = _npad(n)
    npt = npad // NS
    assert nch % RB == 0 and npt % CH == 0

    def body(ei_hbm, out_hbm, ones_s, ones_d, zbuf, sbufs, dbufs, acc,
             isems, ssems, dsems):
        c = lax.axis_index("c")
        s = lax.axis_index("s")
        wid = s * NC + c
        e0 = wid * ept

        lane = lax.iota(jnp.int32, 16)
        one = jnp.ones((LW,), jnp.float32)
        zero = jnp.zeros((LW,), jnp.float32)

        def fill(i, carry):
            ones_s[i] = jnp.where(lane < 8, one, zero)
            ones_d[i] = jnp.where(lane < 8, zero, one)
            zbuf[i] = zero
            return carry

        lax.fori_loop(0, CH, fill, 0)

        r0 = s * npt

        def zrow(i, carry):
            pltpu.sync_copy(zbuf, acc.at[pl.ds(r0 + i * CH, CH)])
            return carry

        lax.fori_loop(0, npt // CH, zrow, 0)
        plsc.subcore_barrier()

        def fire_i(gi, b):
            pltpu.async_copy(ei_hbm.at[pl.ds(e0 + gi * CH, CH)], sbufs[b],
                             isems[b])
            pltpu.async_copy(ei_hbm.at[pl.ds(e + e0 + gi * CH, CH)], dbufs[b],
                             isems[b])

        def wait_i(gi, b):
            pltpu.make_async_copy(ei_hbm.at[pl.ds(e0 + gi * CH, CH)],
                                  sbufs[b], isems[b]).wait()
            pltpu.make_async_copy(ei_hbm.at[pl.ds(e + e0 + gi * CH, CH)],
                                  dbufs[b], isems[b]).wait()

        def fire_s(gi, b):
            pltpu.async_copy(ones_s, acc.at[sbufs[b]], ssems[b], add=True)
            pltpu.async_copy(ones_d, acc.at[dbufs[b]], dsems[b], add=True)

        def wait_s(gi, b):
            pltpu.make_async_copy(ones_s, acc.at[sbufs[b]], ssems[b]).wait()
            pltpu.make_async_copy(ones_d, acc.at[dbufs[b]], dsems[b]).wait()

        for b in range(RB):
            fire_i(b, b)

        def grp(gg, carry):
            g = RB * gg
            for b in range(RB):
                wait_i(g + b, b)
                fire_s(g + b, b)
            for b in range(RB):
                wait_s(g + b, b)
                fire_i(g + RB + b, b)
            return carry

        lax.fori_loop(0, nch // RB - 1, grp, 0)
        ge = nch - RB
        for b in range(RB):
            wait_i(ge + b, b)
            fire_s(ge + b, b)
        for b in range(RB):
            wait_s(ge + b, b)
        plsc.subcore_barrier()

        pltpu.sync_copy(acc.at[pl.ds(r0, npt)], out_hbm.at[c, pl.ds(r0, npt)])

    def wrapped(ei_hbm, out_hbm, ones_s, ones_d, zbuf, *rest):
        sbufs = rest[:RB]
        dbufs = rest[RB:2 * RB]
        acc = rest[2 * RB]
        isems = rest[2 * RB + 1:3 * RB + 1]
        ssems = rest[3 * RB + 1:4 * RB + 1]
        dsems = rest[4 * RB + 1:]
        return body(ei_hbm, out_hbm, ones_s, ones_d, zbuf, sbufs, dbufs, acc,
                    isems, ssems, dsems)

    f = pl.kernel(
        wrapped,
        out_type=jax.ShapeDtypeStruct((NC, npad, LW), jnp.float32),
        mesh=_mesh(),
        compiler_params=pltpu.CompilerParams(use_tc_tiling_on_sc=False),
        scratch_types=[
            pltpu.VMEM((CH, LW), jnp.float32),
            pltpu.VMEM((CH, LW), jnp.float32),
            pltpu.VMEM((CH, LW), jnp.float32),
        ] + [pltpu.VMEM((CH,), jnp.int32) for _ in range(2 * RB)]
          + [pltpu.VMEM_SHARED((npad, LW), jnp.float32)]
          + [pltpu.SemaphoreType.DMA for _ in range(3 * RB)],
    )
    return f(edge_index)


def _aggregate_sc(t, edge_index, n, d, tc_tiling, dtype):
    e = edge_index.shape[0] // 2
    ept = e // NW
    nch = ept // ACH
    npad = _npad(n)
    npt = npad // NS
    assert nch % RB == 0 and npt % ACH == 0

    def body(t_hbm, ei_hbm, out_hbm, sidx, didx, rows, acc,
             gsems, dsems, ssems):
        c = lax.axis_index("c")
        s = lax.axis_index("s")
        wid = s * NC + c
        e0 = wid * ept

        pltpu.sync_copy(ei_hbm.at[pl.ds(e0, ept)], sidx)

        vw = LW if dtype == jnp.float32 else 2 * LW

        def zfill(i, carry):
            for j in range(d // vw):
                rows[0][i, pl.ds(j * vw, vw)] = jnp.zeros((vw,), dtype)
            return carry

        lax.fori_loop(0, ACH, zfill, 0)

        r0 = s * npt

        def zrow(i, carry):
            pltpu.sync_copy(rows[0], acc.at[pl.ds(r0 + i * ACH, ACH)])
            return carry

        lax.fori_loop(0, npt // ACH, zrow, 0)
        plsc.subcore_barrier()

        def fire_g(gi, b):
            pltpu.async_copy(t_hbm.at[sidx.at[pl.ds(gi * ACH, ACH)]], rows[b],
                             gsems[b])
            pltpu.async_copy(ei_hbm.at[pl.ds(e + e0 + gi * ACH, ACH)], didx[b],
                             dsems[b])

        def wait_g(gi, b):
            pltpu.make_async_copy(t_hbm.at[sidx.at[pl.ds(gi * ACH, ACH)]],
                                  rows[b], gsems[b]).wait()
            pltpu.make_async_copy(ei_hbm.at[pl.ds(e + e0 + gi * ACH, ACH)],
                                  didx[b], dsems[b]).wait()

        def fire_s(gi, b):
            pltpu.async_copy(rows[b], acc.at[didx[b]], ssems[b], add=True)

        def wait_s(gi, b):
            pltpu.make_async_copy(rows[b], acc.at[didx[b]], ssems[b]).wait()

        for b in range(RB):
            fire_g(b, b)

        def grp(gg, carry):
            g = RB * gg
            for b in range(RB):
                wait_g(g + b, b)
                fire_s(g + b, b)
            for b in range(RB):
                wait_s(g + b, b)
                fire_g(g + RB + b, b)
            return carry

        lax.fori_loop(0, nch // RB - 1, grp, 0)
        ge = nch - RB
        for b in range(RB):
            wait_g(ge + b, b)
            fire_s(ge + b, b)
        for b in range(RB):
            wait_s(ge + b, b)
        plsc.subcore_barrier()

        pltpu.sync_copy(acc.at[pl.ds(r0, npt)], out_hbm.at[c, pl.ds(r0, npt)])

    def wrapped(t_hbm, ei_hbm, out_hbm, sidx, *rest):
        didx = rest[:RB]
        rows = rest[RB:2 * RB]
        acc = rest[2 * RB]
        gsems = rest[2 * RB + 1:3 * RB + 1]
        dsems = rest[3 * RB + 1:4 * RB + 1]
        ssems = rest[4 * RB + 1:]
        return body(t_hbm, ei_hbm, out_hbm, sidx, didx, rows, acc,
                    gsems, dsems, ssems)

    f = pl.kernel(
        wrapped,
        out_type=jax.ShapeDtypeStruct((NC, npad, d), dtype),
        mesh=_mesh(),
        compiler_params=pltpu.CompilerParams(use_tc_tiling_on_sc=tc_tiling),
        scratch_types=[pltpu.VMEM((ept,), jnp.int32)]
          + [pltpu.VMEM((ACH,), jnp.int32) for _ in range(RB)]
          + [pltpu.VMEM((ACH, d), dtype) for _ in range(RB)]
          + [pltpu.VMEM_SHARED((npad, d), dtype)]
          + [pltpu.SemaphoreType.DMA for _ in range(3 * RB)],
    )
    return f(t, edge_index)


def _k1_body(x_ref, w1_ref, degp_ref, t1_ref, ns_ref, nd_ref):
    dp = degp_ref[...]
    deg_out = dp[0, :, 0] + dp[1, :, 0]
    deg_in = dp[0, :, 8] + dp[1, :, 8]
    ns = jnp.where(deg_out > 0, lax.rsqrt(jnp.maximum(deg_out, 1.0)), 0.0)
    nd = jnp.where(deg_in > 0, lax.rsqrt(jnp.maximum(deg_in, 1.0)), 0.0)
    t1 = jnp.dot(x_ref[...], w1_ref[...], preferred_element_type=jnp.float32)
    t1_ref[...] = (t1 * ns[:, None]).astype(t1_ref.dtype)
    ns_ref[...] = ns[:, None]
    nd_ref[...] = nd[:, None]


def _k3_body(ap_ref, nd_ref, b1_ref, w2_ref, ns_ref, t2_ref):
    a = ap_ref[0].astype(jnp.float32) + ap_ref[1].astype(jnp.float32)
    h = jnp.maximum(a * nd_ref[...] + b1_ref[...], 0.0)
    t2 = jnp.dot(h, w2_ref[...], preferred_element_type=jnp.float32)
    t2_ref[...] = (t2 * ns_ref[...]).astype(t2_ref.dtype)


def _k5_body(ap_ref, nd_ref, b2_ref, o_ref):
    a = ap_ref[0].astype(jnp.float32) + ap_ref[1].astype(jnp.float32)
    o_ref[...] = a * nd_ref[...] + b2_ref[...]


def kernel(x, edge_index, W1, b1, W2, b2):
    n, d_in = x.shape
    d_h = W1.shape[1]
    d_out = W2.shape[1]
    ei_flat = edge_index.reshape(-1)

    degp = _degrees_sc(ei_flat, n)

    R = 1000
    grid = (n // R,)

    t1, nsrc, ndst = pl.pallas_call(
        _k1_body,
        grid=grid,
        in_specs=[
            pl.BlockSpec((R, d_in), lambda i: (i, 0)),
            pl.BlockSpec((d_in, d_h), lambda i: (0, 0)),
            pl.BlockSpec((NC, R, LW), lambda i: (0, i, 0)),
        ],
        out_specs=[
            pl.BlockSpec((R, d_h), lambda i: (i, 0)),
            pl.BlockSpec((R, 1), lambda i: (i, 0)),
            pl.BlockSpec((R, 1), lambda i: (i, 0)),
        ],
        out_shape=[
            jax.ShapeDtypeStruct((n, d_h), jnp.bfloat16),
            jax.ShapeDtypeStruct((n, 1), jnp.float32),
            jax.ShapeDtypeStruct((n, 1), jnp.float32),
        ],
    )(x, W1, degp)

    agg1 = _aggregate_sc(t1, ei_flat, n, d_h, tc_tiling=False, dtype=jnp.bfloat16)

    t2 = pl.pallas_call(
        _k3_body,
        grid=grid,
        in_specs=[
            pl.BlockSpec((NC, R, d_h), lambda i: (0, i, 0)),
            pl.BlockSpec((R, 1), lambda i: (i, 0)),
            pl.BlockSpec((1, d_h), lambda i: (0, 0)),
            pl.BlockSpec((d_h, d_out), lambda i: (0, 0)),
            pl.BlockSpec((R, 1), lambda i: (i, 0)),
        ],
        out_specs=pl.BlockSpec((R, d_out), lambda i: (i, 0)),
        out_shape=jax.ShapeDtypeStruct((n, d_out), jnp.bfloat16),
    )(agg1, ndst, b1[None, :], W2, nsrc)

    agg2 = _aggregate_sc(t2, ei_flat, n, d_out, tc_tiling=False, dtype=jnp.bfloat16)

    out = pl.pallas_call(
        _k5_body,
        grid=grid,
        in_specs=[
            pl.BlockSpec((NC, R, d_out), lambda i: (0, i, 0)),
            pl.BlockSpec((R, 1), lambda i: (i, 0)),
            pl.BlockSpec((1, d_out), lambda i: (0, 0)),
        ],
        out_specs=pl.BlockSpec((R, d_out), lambda i: (i, 0)),
        out_shape=jax.ShapeDtypeStruct((n, d_out), jnp.float32),
    )(agg2, ndst, b2[None, :])

    return out
